# baseline (speedup 1.0000x reference)
import functools

import jax
import jax.numpy as jnp
from jax import lax
from jax.experimental import pallas as pl
from jax.experimental.pallas import tpu as pltpu
from jax.experimental.pallas import tpu_sc as plsc

L = 16
NS = 16
CHUNK = 128


def _make_sc_loss(T: int, V: int):
    tpw = T // NS
    nchunks = tpw // CHUNK
    mesh = plsc.VectorSubcoreMesh(
        core_axis_name="c", subcore_axis_name="s", num_cores=1)

    @functools.partial(
        pl.kernel,
        out_type=[
            jax.ShapeDtypeStruct((NS, L), jnp.float32),
            jax.ShapeDtypeStruct((NS, L), jnp.float32),
            jax.ShapeDtypeStruct((L,), jnp.float32),
        ],
        mesh=mesh,
        scratch_types=[
            pltpu.VMEM((tpw,), jnp.int32),
            pltpu.VMEM((nchunks, CHUNK), jnp.int32),
            pltpu.VMEM((nchunks, CHUNK), jnp.float32),
            pltpu.VMEM((L,), jnp.float32),
            pltpu.VMEM((NS, L), jnp.float32),
            pltpu.VMEM((NS, L), jnp.float32),
            pltpu.SemaphoreType.DMA,
        ],
    )
    def loss_kernel(logp_hbm, tgt_hbm, sums_hbm, cnts_hbm, final_hbm,
                    tgt_v, idx_v, vals_v, stage_v, allsum_v, allcnt_v, sem):
        w = lax.axis_index("s")
        base = w * tpw
        pltpu.sync_copy(tgt_hbm.at[pl.ds(base, tpw)], tgt_v)

        lanes = lax.iota(jnp.int32, L)
        for i in range(tpw // L):
            t = tgt_v[pl.ds(i * L, L)]
            tok = base + i * L + lanes
            idx_v[i * L // CHUNK, pl.ds((i * L) % CHUNK, L)] = tok * V + t

        copies = [
            pltpu.async_copy(logp_hbm.at[idx_v.at[j]], vals_v.at[j], sem)
            for j in range(nchunks)
        ]
        for c in copies:
            c.wait()

        acc = jnp.zeros((L,), jnp.float32)
        cnt = jnp.zeros((L,), jnp.float32)
        for i in range(tpw // L):
            t = tgt_v[pl.ds(i * L, L)]
            v = vals_v[i * L // CHUNK, pl.ds((i * L) % CHUNK, L)]
            m = t > 1
            acc = acc + jnp.where(m, -v, 0.0)
            cnt = cnt + jnp.where(m, 1.0, 0.0)

        stage_v[...] = acc
        pltpu.sync_copy(stage_v, sums_hbm.at[w])
        stage_v[...] = cnt
        pltpu.sync_copy(stage_v, cnts_hbm.at[w])
        plsc.subcore_barrier()

        @pl.when(w == 0)
        def _():
            pltpu.sync_copy(sums_hbm, allsum_v)
            pltpu.sync_copy(cnts_hbm, allcnt_v)
            tot = jnp.zeros((L,), jnp.float32)
            ctot = jnp.zeros((L,), jnp.float32)
            for r in range(NS):
                tot = tot + allsum_v[r, :]
                ctot = ctot + allcnt_v[r, :]

            def lane_total(x):
                for k in (1, 2, 4, 8):
                    x = x + x.at[lanes ^ k].get(mode="promise_in_bounds")
                return x

            stage_v[...] = lane_total(tot) / lane_total(ctot)
            pltpu.sync_copy(stage_v, final_hbm)

    return loss_kernel


def kernel(logp, target):
    B, S, V = logp.shape
    target = target[:, :S]
    logp_flat = logp.reshape(-1)
    tgt_flat = target.reshape(-1).astype(jnp.int32)
    T = B * S
    _, _, final = _make_sc_loss(T, V)(logp_flat, tgt_flat)
    return final[0]


# speedup vs baseline: 8.9684x; 8.9684x over previous
import functools

import jax
import jax.numpy as jnp
from jax import lax
from jax.experimental import pallas as pl
from jax.experimental.pallas import tpu as pltpu
from jax.experimental.pallas import tpu_sc as plsc

L = 16
NS = 16
CHUNK = 128


def _make_sc_loss(T: int, V: int):
    tpw = T // NS
    nchunks = tpw // CHUNK
    ctiles = V // 128
    mesh = plsc.VectorSubcoreMesh(
        core_axis_name="c", subcore_axis_name="s", num_cores=1)

    @functools.partial(
        pl.kernel,
        out_type=[
            jax.ShapeDtypeStruct((NS, L), jnp.float32),
            jax.ShapeDtypeStruct((NS, L), jnp.float32),
            jax.ShapeDtypeStruct((L,), jnp.float32),
        ],
        mesh=mesh,
        scratch_types=[
            pltpu.VMEM((tpw,), jnp.int32),
            pltpu.VMEM((nchunks, CHUNK), jnp.int32),
            pltpu.VMEM((tpw, 128), jnp.float32),
            pltpu.VMEM((L,), jnp.float32),
            pltpu.VMEM((NS, L), jnp.float32),
            pltpu.VMEM((NS, L), jnp.float32),
            pltpu.SemaphoreType.DMA,
        ],
        compiler_params=pltpu.CompilerParams(
            use_tc_tiling_on_sc=False, needs_layout_passes=False),
    )
    def loss_kernel(logp_hbm, tgt_hbm, sums_hbm, cnts_hbm, final_hbm,
                    tgt_v, idx_v, rows_v, stage_v, allsum_v, allcnt_v, sem):
        w = lax.axis_index("s")
        base = w * tpw
        pltpu.sync_copy(tgt_hbm.at[pl.ds(base, tpw)], tgt_v)

        lanes = lax.iota(jnp.int32, L)
        copies = []
        for j in range(nchunks):
            for i in range(CHUNK // L):
                t = base + j * CHUNK + i * L + lanes
                c = tgt_v[pl.ds(j * CHUNK + i * L, L)]
                idx_v[j, pl.ds(i * L, L)] = (
                    ((t >> 3) * ctiles + (c >> 7)) * 8 + (t & 7))
            copies.append(pltpu.async_copy(
                logp_hbm.at[idx_v.at[j]],
                rows_v.at[pl.ds(j * CHUNK, CHUNK)], sem))

        acc = jnp.zeros((L,), jnp.float32)
        cnt = jnp.zeros((L,), jnp.float32)
        for j in range(nchunks):
            copies[j].wait()
            for i in range(CHUNK // L):
                off = j * CHUNK + i * L
                c = tgt_v[pl.ds(off, L)]
                v = plsc.load_gather(rows_v, [off + lanes, c & 127])
                m = c > 1
                acc = acc + jnp.where(m, -v, 0.0)
                cnt = cnt + jnp.where(m, 1.0, 0.0)

        stage_v[...] = acc
        pltpu.sync_copy(stage_v, sums_hbm.at[w])
        stage_v[...] = cnt
        pltpu.sync_copy(stage_v, cnts_hbm.at[w])
        plsc.subcore_barrier()

        @pl.when(w == 0)
        def _():
            pltpu.sync_copy(sums_hbm, allsum_v)
            pltpu.sync_copy(cnts_hbm, allcnt_v)
            tot = jnp.zeros((L,), jnp.float32)
            ctot = jnp.zeros((L,), jnp.float32)
            for r in range(NS):
                tot = tot + allsum_v[r, :]
                ctot = ctot + allcnt_v[r, :]

            def lane_total(x):
                for k in (1, 2, 4, 8):
                    x = x + x.at[lanes ^ k].get(mode="promise_in_bounds")
                return x

            stage_v[...] = lane_total(tot) / lane_total(ctot)
            pltpu.sync_copy(stage_v, final_hbm)

    return loss_kernel


def kernel(logp, target):
    B, S, V = logp.shape
    target = target[:, :S]
    T = B * S
    lp_r = (logp.reshape(T // 8, 8, V // 128, 128)
            .transpose(0, 2, 1, 3)
            .reshape(T * V // 128, 128))
    tgt = target.reshape(-1).astype(jnp.int32)
    _, _, final = _make_sc_loss(T, V)(lp_r, tgt)
    return final[0]


# speedup vs baseline: 9.7868x; 1.0913x over previous
import functools

import jax
import jax.numpy as jnp
from jax import lax
from jax.experimental import pallas as pl
from jax.experimental.pallas import tpu as pltpu
from jax.experimental.pallas import tpu_sc as plsc

L = 16
NS = 16
CHUNK = 128


def _make_sc_loss(T: int, V: int):
    tpw = T // NS
    nchunks = tpw // CHUNK
    ctiles = V // 128
    mesh = plsc.VectorSubcoreMesh(
        core_axis_name="c", subcore_axis_name="s", num_cores=1)

    @functools.partial(
        pl.kernel,
        out_type=[
            jax.ShapeDtypeStruct((NS, L), jnp.float32),
            jax.ShapeDtypeStruct((NS, L), jnp.float32),
            jax.ShapeDtypeStruct((L,), jnp.float32),
        ],
        mesh=mesh,
        scratch_types=[
            pltpu.VMEM((tpw,), jnp.int32),
            pltpu.VMEM((nchunks, CHUNK), jnp.int32),
            pltpu.VMEM((tpw, 16), jnp.float32),
            pltpu.VMEM((L,), jnp.float32),
            pltpu.VMEM((NS, L), jnp.float32),
            pltpu.VMEM((NS, L), jnp.float32),
            pltpu.SemaphoreType.DMA,
        ],
        compiler_params=pltpu.CompilerParams(
            use_tc_tiling_on_sc=False, needs_layout_passes=False),
    )
    def loss_kernel(logp_hbm, tgt_hbm, sums_hbm, cnts_hbm, final_hbm,
                    tgt_v, idx_v, rows_v, stage_v, allsum_v, allcnt_v, sem):
        w = lax.axis_index("s")
        base = w * tpw
        pltpu.sync_copy(tgt_hbm.at[pl.ds(base, tpw)], tgt_v)

        lanes = lax.iota(jnp.int32, L)
        copies = []
        for j in range(nchunks):
            for i in range(CHUNK // L):
                t = base + j * CHUNK + i * L + lanes
                c = tgt_v[pl.ds(j * CHUNK + i * L, L)]
                krow = ((t >> 3) * ctiles + (c >> 7)) * 8 + (t & 7)
                idx_v[j, pl.ds(i * L, L)] = krow * 8 + ((c >> 4) & 7)
            copies.append(pltpu.async_copy(
                logp_hbm.at[idx_v.at[j]],
                rows_v.at[pl.ds(j * CHUNK, CHUNK)], sem))

        acc = jnp.zeros((L,), jnp.float32)
        cnt = jnp.zeros((L,), jnp.float32)
        for j in range(nchunks):
            copies[j].wait()
            for i in range(CHUNK // L):
                off = j * CHUNK + i * L
                c = tgt_v[pl.ds(off, L)]
                v = plsc.load_gather(rows_v, [off + lanes, c & 15])
                m = c > 1
                acc = acc + jnp.where(m, -v, 0.0)
                cnt = cnt + jnp.where(m, 1.0, 0.0)

        stage_v[...] = acc
        pltpu.sync_copy(stage_v, sums_hbm.at[w])
        stage_v[...] = cnt
        pltpu.sync_copy(stage_v, cnts_hbm.at[w])
        plsc.subcore_barrier()

        @pl.when(w == 0)
        def _():
            pltpu.sync_copy(sums_hbm, allsum_v)
            pltpu.sync_copy(cnts_hbm, allcnt_v)
            tot = jnp.zeros((L,), jnp.float32)
            ctot = jnp.zeros((L,), jnp.float32)
            for r in range(NS):
                tot = tot + allsum_v[r, :]
                ctot = ctot + allcnt_v[r, :]

            def lane_total(x):
                for k in (1, 2, 4, 8):
                    x = x + x.at[lanes ^ k].get(mode="promise_in_bounds")
                return x

            stage_v[...] = lane_total(tot) / lane_total(ctot)
            pltpu.sync_copy(stage_v, final_hbm)

    return loss_kernel


def kernel(logp, target):
    B, S, V = logp.shape
    target = target[:, :S]
    T = B * S
    lp_r = (logp.reshape(T // 8, 8, V // 128, 128)
            .transpose(0, 2, 1, 3)
            .reshape(T * V // 16, 16))
    tgt = target.reshape(-1).astype(jnp.int32)
    _, _, final = _make_sc_loss(T, V)(lp_r, tgt)
    return final[0]


# speedup vs baseline: 10.2360x; 1.0459x over previous
import functools

import jax
import jax.numpy as jnp
from jax import lax
from jax.experimental import pallas as pl
from jax.experimental.pallas import tpu as pltpu
from jax.experimental.pallas import tpu_sc as plsc

L = 16
NS = 16
CHUNK = 128


def _make_sc_loss(T: int, V: int):
    tpw = T // NS
    nchunks = tpw // CHUNK
    ctiles = V // 128
    mesh = plsc.VectorSubcoreMesh(
        core_axis_name="c", subcore_axis_name="s", num_cores=1)

    @functools.partial(
        pl.kernel,
        out_type=[
            jax.ShapeDtypeStruct((NS, L), jnp.float32),
            jax.ShapeDtypeStruct((NS, L), jnp.float32),
            jax.ShapeDtypeStruct((L,), jnp.float32),
        ],
        mesh=mesh,
        scratch_types=[
            pltpu.VMEM((tpw,), jnp.int32),
            pltpu.VMEM((nchunks, CHUNK), jnp.int32),
            pltpu.VMEM((tpw, 16), jnp.float32),
            pltpu.VMEM((L,), jnp.float32),
            pltpu.VMEM((NS, L), jnp.float32),
            pltpu.VMEM((NS, L), jnp.float32),
            pltpu.VMEM_SHARED((NS, L), jnp.float32),
            pltpu.VMEM_SHARED((NS, L), jnp.float32),
            pltpu.SemaphoreType.DMA,
        ],
        compiler_params=pltpu.CompilerParams(
            use_tc_tiling_on_sc=False, needs_layout_passes=False),
    )
    def loss_kernel(logp_hbm, tgt_hbm, sums_hbm, cnts_hbm, final_hbm,
                    tgt_v, idx_v, rows_v, stage_v, allsum_v, allcnt_v,
                    shsum_v, shcnt_v, sem):
        w = lax.axis_index("s")
        base = w * tpw
        pltpu.sync_copy(tgt_hbm.at[pl.ds(base, tpw)], tgt_v)

        lanes = lax.iota(jnp.int32, L)
        copies = []
        for j in range(nchunks):
            for i in range(CHUNK // L):
                t = base + j * CHUNK + i * L + lanes
                c = tgt_v[pl.ds(j * CHUNK + i * L, L)]
                krow = ((t >> 3) * ctiles + (c >> 7)) * 8 + (t & 7)
                idx_v[j, pl.ds(i * L, L)] = krow * 8 + ((c >> 4) & 7)
            copies.append(pltpu.async_copy(
                logp_hbm.at[idx_v.at[j]],
                rows_v.at[pl.ds(j * CHUNK, CHUNK)], sem))

        acc = jnp.zeros((L,), jnp.float32)
        cnt = jnp.zeros((L,), jnp.float32)
        for j in range(nchunks):
            copies[j].wait()
            for i in range(CHUNK // L):
                off = j * CHUNK + i * L
                c = tgt_v[pl.ds(off, L)]
                v = plsc.load_gather(rows_v, [off + lanes, c & 15])
                m = c > 1
                acc = acc + jnp.where(m, -v, 0.0)
                cnt = cnt + jnp.where(m, 1.0, 0.0)

        stage_v[...] = acc
        pltpu.sync_copy(stage_v, shsum_v.at[w])
        stage_v[...] = cnt
        pltpu.sync_copy(stage_v, shcnt_v.at[w])
        plsc.subcore_barrier()

        @pl.when(w == 0)
        def _():
            pltpu.sync_copy(shsum_v, allsum_v)
            pltpu.sync_copy(shcnt_v, allcnt_v)
            tot = jnp.zeros((L,), jnp.float32)
            ctot = jnp.zeros((L,), jnp.float32)
            for r in range(NS):
                tot = tot + allsum_v[r, :]
                ctot = ctot + allcnt_v[r, :]

            def lane_total(x):
                for k in (1, 2, 4, 8):
                    x = x + x.at[lanes ^ k].get(mode="promise_in_bounds")
                return x

            stage_v[...] = lane_total(tot) / lane_total(ctot)
            pltpu.sync_copy(stage_v, final_hbm)

    return loss_kernel


def kernel(logp, target):
    B, S, V = logp.shape
    target = target[:, :S]
    T = B * S
    lp_r = (logp.reshape(T // 8, 8, V // 128, 128)
            .transpose(0, 2, 1, 3)
            .reshape(T * V // 16, 16))
    tgt = target.reshape(-1).astype(jnp.int32)
    _, _, final = _make_sc_loss(T, V)(lp_r, tgt)
    return final[0]
